# Initial kernel scaffold; baseline (speedup 1.0000x reference)
import jax
import jax.numpy as jnp
from jax.experimental import pallas as pl


def kernel(target_features, form_features, role_features, form_neighbors, role_neighbors, form_binds_ei, form_binds_y, role_binds_ei, role_binds_y, drug_features, params):
    raise NotImplementedError("write your pallas kernel here")


# speedup vs baseline: 16.6555x; 16.6555x over previous
import functools
import math

import jax
import jax.numpy as jnp
from jax import lax
from jax.experimental import pallas as pl
from jax.experimental.pallas import tpu as pltpu
from jax.experimental.pallas import tpu_sc as plsc

P = 512
D = 256
NN = 1024
NE = 32768
ND = 10000
BASELINE = 6.0
INV_SQRT_D = 1.0 / math.sqrt(D)

NC, NS = 2, 16
EPT = NE // NS
NCH = EPT // 16
NDP = 10240
SEG_PT = NN // NS
AW = 16


def _stage_a_body(tgt, prW1, prb1, pra, prW2, prb2, qW, qb, kW, kb, drug,
                  z_out, sc_out):
    t = tgt[...]
    h = jnp.dot(t, prW1[...]) + prb1[...]
    a = pra[0, 0]
    h = jnp.where(h >= 0.0, h, a * h)
    z = jnp.dot(h, prW2[...]) + prb2[...]
    z_out[...] = z
    q = jnp.dot(z, qW[...]) + qb[...]
    qk = lax.dot_general(q, kW[...], (((1,), (1,)), ((), ())))
    c = jnp.sum(kb[...] * q)
    raw = jnp.sum(drug[...] * qk, axis=1, keepdims=True)
    sc_out[...] = (raw + c) * INV_SQRT_D


def _stage_a(tgt, p, drug):
    return pl.pallas_call(
        _stage_a_body,
        out_shape=(
            jax.ShapeDtypeStruct((1, P), jnp.float32),
            jax.ShapeDtypeStruct((ND, 1), jnp.float32),
        ),
    )(tgt, p["pr_W1"], p["pr_b1"].reshape(1, P), p["pr_a"].reshape(1, 1),
      p["pr_W2"], p["pr_b2"].reshape(1, P), p["q_W"], p["q_b"].reshape(1, D),
      p["k_W"], p["k_b"].reshape(1, D), drug)


def _sc_body(scores_hbm, didx_hbm, src_hbm, y_hbm, drug_hbm,
             out_hbm, aux_hbm,
             scores_v, didx_v, src_v, y_v, selp_v, e16_v, u16_v, seg16_v,
             didx16_v, mx_v, mxall_v, rows_v, acc_v, aux_v, mx_sh, sem):
    cid = lax.axis_index("c")
    sid = lax.axis_index("s")
    lane = lax.broadcasted_iota(jnp.int32, (16,), 0)
    zero16 = jnp.zeros((16,), jnp.float32)
    lo = sid * SEG_PT

    pltpu.sync_copy(scores_hbm, scores_v.at[pl.ds(0, ND)])

    def zrow(r, _):
        for k in range(D // 16):
            acc_v[pl.ds(r * D + k * 16, 16)] = zero16
        aux_v[pl.ds(r * AW, 16)] = zero16
        return 0

    lax.fori_loop(0, SEG_PT, zrow, 0)

    pltpu.sync_copy(didx_hbm.at[cid, pl.ds(sid * EPT, EPT)], didx_v)

    def p1(i, mx):
        dv = didx_v[pl.ds(i * 16, 16)]
        return jnp.maximum(mx, plsc.load_gather(scores_v, [dv]))

    mx = lax.fori_loop(0, NCH, p1, jnp.full((16,), -1e30, jnp.float32))
    mx_v[...] = mx
    pltpu.sync_copy(mx_v, mx_sh.at[sid])
    plsc.subcore_barrier()
    pltpu.sync_copy(mx_sh, mxall_v)
    acc = mxall_v[0, ...]
    for r in range(1, NS):
        acc = jnp.maximum(acc, mxall_v[r, ...])
    M = jnp.max(acc)

    def strip(q, _):
        sb = q * EPT
        pltpu.sync_copy(didx_hbm.at[cid, pl.ds(sb, EPT)], didx_v)
        pltpu.sync_copy(src_hbm.at[cid, pl.ds(sb, EPT)], src_v)
        pltpu.sync_copy(y_hbm.at[cid, pl.ds(sb, EPT)], y_v)

        def compact(i, nsel):
            sv = src_v[pl.ds(i * 16, 16)]
            seg = sv - lo
            m = (seg >= 0) & (seg < SEG_PT)
            pos = jnp.full((16,), i * 16, jnp.int32) + lane
            offs = plsc.cumsum(jnp.where(m, 1, 0))
            wpos = jnp.full((16,), nsel - 1, jnp.int32) + offs
            plsc.store_scatter(selp_v, [wpos], (seg << 16) + pos, mask=m)
            return nsel + jnp.max(offs)

        nsel = lax.fori_loop(0, NCH, compact, jnp.int32(0))

        def chunk(j, _):
            rem = nsel - j * 16
            valid = lane < jnp.full((16,), rem, jnp.int32)
            sp = selp_v[pl.ds(j * 16, 16)]
            sp = jnp.where(valid, sp, 0)
            seg = sp >> 16
            pos = sp & 65535
            dv = plsc.load_gather(didx_v, [pos])
            yv = plsc.load_gather(y_v, [pos])
            didx16_v[...] = dv
            dvm = didx16_v[...]
            sv = plsc.load_gather(scores_v, [dvm])
            e = jnp.where(valid, jnp.exp(sv - M), 0.0)
            u = e * (yv - BASELINE)
            pltpu.async_copy(drug_hbm.at[didx16_v], rows_v, sem).wait()
            for r in range(16):
                gidx = jnp.full((16,), r, jnp.int32)
                u_s = jnp.take_along_axis(u, gidx, axis=0)
                e_s = jnp.take_along_axis(e, gidx, axis=0)
                g_s = jnp.take_along_axis(seg, gidx, axis=0)
                gb = g_s * D + lane
                for k in range(D // 16):
                    v = rows_v[r, pl.ds(k * 16, 16)] * u_s
                    plsc.addupdate_scatter(acc_v, [gb + k * 16], v)
                tail = jnp.where(lane == 0, e_s,
                                 jnp.where(lane == 1, u_s, 0.0))
                plsc.addupdate_scatter(aux_v, [g_s * AW + lane], tail)
            return 0

        lax.fori_loop(0, (nsel + 15) // 16, chunk, 0)
        return 0

    lax.fori_loop(0, NS, strip, 0)

    pltpu.sync_copy(acc_v, out_hbm.at[cid, pl.ds(lo * D, SEG_PT * D)])
    pltpu.sync_copy(aux_v, aux_hbm.at[cid, pl.ds(lo * AW, SEG_PT * AW)])


def _stage_b(scores, didx2, src2, y2, drug):
    mesh = plsc.VectorSubcoreMesh(core_axis_name="c", subcore_axis_name="s")
    f = pl.kernel(
        _sc_body,
        out_type=(
            jax.ShapeDtypeStruct((2, NN * D), jnp.float32),
            jax.ShapeDtypeStruct((2, NN * AW), jnp.float32),
        ),
        mesh=mesh,
        compiler_params=pltpu.CompilerParams(needs_layout_passes=False),
        scratch_types=[
            pltpu.VMEM((NDP,), jnp.float32),
            pltpu.VMEM((EPT,), jnp.int32),
            pltpu.VMEM((EPT,), jnp.int32),
            pltpu.VMEM((EPT,), jnp.float32),
            pltpu.VMEM((EPT + 128,), jnp.int32),
            pltpu.VMEM((16,), jnp.float32),
            pltpu.VMEM((16,), jnp.float32),
            pltpu.VMEM((16,), jnp.int32),
            pltpu.VMEM((16,), jnp.int32),
            pltpu.VMEM((16,), jnp.float32),
            pltpu.VMEM((NS, 16), jnp.float32),
            pltpu.VMEM((16, D), jnp.float32),
            pltpu.VMEM((SEG_PT * D,), jnp.float32),
            pltpu.VMEM((SEG_PT * AW,), jnp.float32),
            pltpu.VMEM_SHARED((NS, 16), jnp.float32),
            pltpu.SemaphoreType.DMA,
        ],
    )
    Sx, Ax = f(scores, didx2, src2, y2, drug)
    return Sx.reshape(2, NN, D), Ax.reshape(2, NN, AW)


def _stage_c_body(form, role, z, prW1, prb1, pra, prW2, prb2,
                  a1W, a1b, a2W, a2b, a3W, a3b, lemb, fa_out, ra_out):
    a = pra[0, 0]

    def refine(x):
        h = jnp.dot(x, prW1[...]) + prb1[...]
        h = jnp.where(h >= 0.0, h, a * h)
        return jnp.dot(h, prW2[...]) + prb2[...]

    zrow = z[...]
    Wz = a1W[0:P, :]
    Wn = a1W[P:2 * P, :]
    Wl = a1W[2 * P:, :]
    le = lemb[...]

    def attn(xr, li, out):
        bse = jnp.dot(zrow, Wz) + jnp.dot(le[li:li + 1, :], Wl) + a1b[...]
        h = jnp.dot(xr, Wn) + bse
        h = jnp.where(h >= 0.0, h, 0.2 * h)
        h = jnp.dot(h, a2W[...]) + a2b[...]
        h = jnp.where(h >= 0.0, h, 0.2 * h)
        out[...] = jnp.dot(h, a3W[...]) + a3b[...]

    attn(refine(form[...]), 0, fa_out)
    attn(refine(role[...]), 1, ra_out)


def _stage_c(form, role, z, p):
    return pl.pallas_call(
        _stage_c_body,
        out_shape=(
            jax.ShapeDtypeStruct((NN, 1), jnp.float32),
            jax.ShapeDtypeStruct((NN, 1), jnp.float32),
        ),
    )(form, role, z, p["pr_W1"], p["pr_b1"].reshape(1, P),
      p["pr_a"].reshape(1, 1), p["pr_W2"], p["pr_b2"].reshape(1, P),
      p["a1_W"], p["a1_b"].reshape(1, 128), p["a2_W"],
      p["a2_b"].reshape(1, 64), p["a3_W"], p["a3_b"].reshape(1, 1),
      p["layer_emb"])


def _stage_d_body(Sx, Ax, vW, vb, fa, ra, z, imW1, imb1, ima, imW2, imb2,
                  lng, lnb, zr_out, fm_out, rm_out):
    vWm = vW[...]
    vbr = vb[...]

    def finish(S, A):
        den = A[:, 0:1]
        dsafe = jnp.where(den > 0.0, den, 1.0)
        t = A[:, 1:2]
        return (jnp.dot(S / dsafe, vWm) + (t / dsafe) * vbr)

    fm = finish(Sx[0], Ax[0])
    rm = finish(Sx[1], Ax[1])
    fm_out[...] = fm
    rm_out[...] = rm

    fav = fa[...]
    rav = ra[...]
    m = jnp.maximum(jnp.max(fav), jnp.max(rav))
    wf = jnp.exp(fav - m)
    wr = jnp.exp(rav - m)
    sw = jnp.sum(wf) + jnp.sum(wr)
    vp = (jnp.sum(wf * fm, axis=0, keepdims=True)
          + jnp.sum(wr * rm, axis=0, keepdims=True)) / sw

    h = jnp.dot(vp, imW1[...]) + imb1[...]
    av = ima[0, 0]
    h = jnp.where(h >= 0.0, h, av * h)
    iv = jnp.dot(h, imW2[...]) + imb2[...]
    zi = z[...] + iv
    mu = jnp.mean(zi, axis=-1, keepdims=True)
    var = jnp.mean((zi - mu) ** 2, axis=-1, keepdims=True)
    zr_out[...] = (zi - mu) * lax.rsqrt(var + 1e-5) * lng[...] + lnb[...]


def _stage_d(Sx, Ax, fa, ra, z, p):
    return pl.pallas_call(
        _stage_d_body,
        out_shape=(
            jax.ShapeDtypeStruct((1, P), jnp.float32),
            jax.ShapeDtypeStruct((NN, D), jnp.float32),
            jax.ShapeDtypeStruct((NN, D), jnp.float32),
        ),
    )(Sx, Ax, p["v_W"], p["v_b"].reshape(1, D), fa, ra, z,
      p["im_W1"], p["im_b1"].reshape(1, P), p["im_a"].reshape(1, 1),
      p["im_W2"], p["im_b2"].reshape(1, P), p["ln_g"].reshape(1, P),
      p["ln_b"].reshape(1, P))


def kernel(target_features, form_features, role_features, form_neighbors,
           role_neighbors, form_binds_ei, form_binds_y, role_binds_ei,
           role_binds_y, drug_features, params):
    p = params
    tgt = target_features.reshape(1, P)
    z, scores2d = _stage_a(tgt, p, drug_features)
    scores = scores2d.reshape(ND)

    didx2 = jnp.stack([form_binds_ei[1], role_binds_ei[1]])
    src2 = jnp.stack([form_binds_ei[0], role_binds_ei[0]])
    y2 = jnp.stack([form_binds_y, role_binds_y])
    Sx, Ax = _stage_b(scores, didx2, src2, y2, drug_features)

    fa, ra = _stage_c(form_features, role_features, z, p)
    zr, fm, rm = _stage_d(Sx, Ax, fa, ra, z, p)
    return (zr.reshape(P), fm, rm)


# speedup vs baseline: 16.6621x; 1.0004x over previous
import functools
import math

import jax
import jax.numpy as jnp
from jax import lax
from jax.experimental import pallas as pl
from jax.experimental.pallas import tpu as pltpu
from jax.experimental.pallas import tpu_sc as plsc

P = 512
D = 256
NN = 1024
NE = 32768
ND = 10000
BASELINE = 6.0
INV_SQRT_D = 1.0 / math.sqrt(D)

NC, NS = 2, 16
EPT = NE // NS
NCH = EPT // 16
NDP = 10240
SEG_PT = NN // NS
AW = 16


def _stage_a_body(tgt, prW1, prb1, pra, prW2, prb2, qW, qb, kW, kb, drug,
                  z_out, sc_out):
    t = tgt[...]
    h = jnp.dot(t, prW1[...]) + prb1[...]
    a = pra[0, 0]
    h = jnp.where(h >= 0.0, h, a * h)
    z = jnp.dot(h, prW2[...]) + prb2[...]
    z_out[...] = z
    q = jnp.dot(z, qW[...]) + qb[...]
    qk = lax.dot_general(q, kW[...], (((1,), (1,)), ((), ())))
    c = jnp.sum(kb[...] * q)
    raw = jnp.sum(drug[...] * qk, axis=1, keepdims=True)
    sc_out[...] = (raw + c) * INV_SQRT_D


def _stage_a(tgt, p, drug):
    return pl.pallas_call(
        _stage_a_body,
        out_shape=(
            jax.ShapeDtypeStruct((1, P), jnp.float32),
            jax.ShapeDtypeStruct((ND, 1), jnp.float32),
        ),
    )(tgt, p["pr_W1"], p["pr_b1"].reshape(1, P), p["pr_a"].reshape(1, 1),
      p["pr_W2"], p["pr_b2"].reshape(1, P), p["q_W"], p["q_b"].reshape(1, D),
      p["k_W"], p["k_b"].reshape(1, D), drug)


def _sc_body(scores_hbm, didx_hbm, src_hbm, y_hbm, drug_hbm,
             out_hbm, aux_hbm,
             scores_v, didx_v, src_v, y_v, selp_v,
             didx16_v, mx_v, mxall_v, rows_v, acc_v, aux_v, mx_sh, sem):
    cid = lax.axis_index("c")
    sid = lax.axis_index("s")
    lane = lax.broadcasted_iota(jnp.int32, (16,), 0)
    zero16 = jnp.zeros((16,), jnp.float32)
    lo = sid * SEG_PT

    pltpu.sync_copy(scores_hbm, scores_v.at[pl.ds(0, ND)])

    def zrow(r, _):
        for k in range(D // 16):
            acc_v[pl.ds(r * D + k * 16, 16)] = zero16
        aux_v[pl.ds(r * AW, 16)] = zero16
        return 0

    lax.fori_loop(0, SEG_PT, zrow, 0)

    pltpu.sync_copy(didx_hbm.at[cid, pl.ds(sid * EPT, EPT)], didx_v)

    def p1(i, mx):
        dv = didx_v[pl.ds(i * 16, 16)]
        return jnp.maximum(mx, plsc.load_gather(scores_v, [dv]))

    mx = lax.fori_loop(0, NCH, p1, jnp.full((16,), -1e30, jnp.float32))
    mx_v[...] = mx
    pltpu.sync_copy(mx_v, mx_sh.at[sid])
    plsc.subcore_barrier()
    pltpu.sync_copy(mx_sh, mxall_v)
    acc = mxall_v[0, ...]
    for r in range(1, NS):
        acc = jnp.maximum(acc, mxall_v[r, ...])
    M = jnp.max(acc)

    def strip(q, _):
        sb = q * EPT
        pltpu.sync_copy(didx_hbm.at[cid, pl.ds(sb, EPT)], didx_v)
        pltpu.sync_copy(src_hbm.at[cid, pl.ds(sb, EPT)], src_v)
        pltpu.sync_copy(y_hbm.at[cid, pl.ds(sb, EPT)], y_v)

        def compact(i, nsel):
            sv = src_v[pl.ds(i * 16, 16)]
            seg = sv - lo
            m = (seg >= 0) & (seg < SEG_PT)
            pos = jnp.full((16,), i * 16, jnp.int32) + lane
            offs = plsc.cumsum(jnp.where(m, 1, 0))
            wpos = jnp.full((16,), nsel - 1, jnp.int32) + offs
            plsc.store_scatter(selp_v, [wpos], (seg << 16) + pos, mask=m)
            return nsel + jnp.max(offs)

        nsel = lax.fori_loop(0, NCH, compact, jnp.int32(0))

        def chunk(j, _):
            rem = nsel - j * 16
            valid = lane < jnp.full((16,), rem, jnp.int32)
            sp = selp_v[pl.ds(j * 16, 16)]
            sp = jnp.where(valid, sp, 0)
            seg = sp >> 16
            pos = sp & 65535
            dv = plsc.load_gather(didx_v, [pos])
            yv = plsc.load_gather(y_v, [pos])
            didx16_v[...] = dv
            dvm = didx16_v[...]
            sv = plsc.load_gather(scores_v, [dvm])
            e = jnp.where(valid, jnp.exp(sv - M), 0.0)
            u = e * (yv - BASELINE)
            pltpu.async_copy(drug_hbm.at[didx16_v], rows_v, sem).wait()
            for r in range(16):
                gidx = jnp.full((16,), r, jnp.int32)
                u_s = jnp.take_along_axis(u, gidx, axis=0)
                e_s = jnp.take_along_axis(e, gidx, axis=0)
                g_s = jnp.take_along_axis(seg, gidx, axis=0)
                gb = g_s * D + lane
                for k in range(D // 16):
                    v = rows_v[r, pl.ds(k * 16, 16)] * u_s
                    plsc.addupdate_scatter(acc_v, [gb + k * 16], v)
                tail = jnp.where(lane == 0, e_s,
                                 jnp.where(lane == 1, u_s, 0.0))
                plsc.addupdate_scatter(aux_v, [g_s * AW + lane], tail)
            return 0

        lax.fori_loop(0, (nsel + 15) // 16, chunk, 0)
        return 0

    lax.fori_loop(0, NS, strip, 0)

    pltpu.sync_copy(acc_v, out_hbm.at[cid, pl.ds(lo * D, SEG_PT * D)])
    pltpu.sync_copy(aux_v, aux_hbm.at[cid, pl.ds(lo * AW, SEG_PT * AW)])


def _stage_b(scores, didx2, src2, y2, drug):
    mesh = plsc.VectorSubcoreMesh(core_axis_name="c", subcore_axis_name="s")
    f = pl.kernel(
        _sc_body,
        out_type=(
            jax.ShapeDtypeStruct((2, NN * D), jnp.float32),
            jax.ShapeDtypeStruct((2, NN * AW), jnp.float32),
        ),
        mesh=mesh,
        compiler_params=pltpu.CompilerParams(needs_layout_passes=False),
        scratch_types=[
            pltpu.VMEM((NDP,), jnp.float32),
            pltpu.VMEM((EPT,), jnp.int32),
            pltpu.VMEM((EPT,), jnp.int32),
            pltpu.VMEM((EPT,), jnp.float32),
            pltpu.VMEM((EPT + 128,), jnp.int32),
            pltpu.VMEM((16,), jnp.int32),
            pltpu.VMEM((16,), jnp.float32),
            pltpu.VMEM((NS, 16), jnp.float32),
            pltpu.VMEM((16, D), jnp.float32),
            pltpu.VMEM((SEG_PT * D,), jnp.float32),
            pltpu.VMEM((SEG_PT * AW,), jnp.float32),
            pltpu.VMEM_SHARED((NS, 16), jnp.float32),
            pltpu.SemaphoreType.DMA,
        ],
    )
    Sx, Ax = f(scores, didx2, src2, y2, drug)
    return Sx.reshape(2, NN, D), Ax.reshape(2, NN, AW)


def _stage_c_body(form, role, z, prW1, prb1, pra, prW2, prb2,
                  a1W, a1b, a2W, a2b, a3W, a3b, lemb, fa_out, ra_out):
    a = pra[0, 0]

    def refine(x):
        h = jnp.dot(x, prW1[...]) + prb1[...]
        h = jnp.where(h >= 0.0, h, a * h)
        return jnp.dot(h, prW2[...]) + prb2[...]

    zrow = z[...]
    Wz = a1W[0:P, :]
    Wn = a1W[P:2 * P, :]
    Wl = a1W[2 * P:, :]
    le = lemb[...]

    def attn(xr, li, out):
        bse = jnp.dot(zrow, Wz) + jnp.dot(le[li:li + 1, :], Wl) + a1b[...]
        h = jnp.dot(xr, Wn) + bse
        h = jnp.where(h >= 0.0, h, 0.2 * h)
        h = jnp.dot(h, a2W[...]) + a2b[...]
        h = jnp.where(h >= 0.0, h, 0.2 * h)
        out[...] = jnp.dot(h, a3W[...]) + a3b[...]

    attn(refine(form[...]), 0, fa_out)
    attn(refine(role[...]), 1, ra_out)


def _stage_c(form, role, z, p):
    return pl.pallas_call(
        _stage_c_body,
        out_shape=(
            jax.ShapeDtypeStruct((NN, 1), jnp.float32),
            jax.ShapeDtypeStruct((NN, 1), jnp.float32),
        ),
    )(form, role, z, p["pr_W1"], p["pr_b1"].reshape(1, P),
      p["pr_a"].reshape(1, 1), p["pr_W2"], p["pr_b2"].reshape(1, P),
      p["a1_W"], p["a1_b"].reshape(1, 128), p["a2_W"],
      p["a2_b"].reshape(1, 64), p["a3_W"], p["a3_b"].reshape(1, 1),
      p["layer_emb"])


def _stage_d_body(Sx, Ax, vW, vb, fa, ra, z, imW1, imb1, ima, imW2, imb2,
                  lng, lnb, zr_out, fm_out, rm_out):
    vWm = vW[...]
    vbr = vb[...]

    def finish(S, A):
        den = A[:, 0:1]
        dsafe = jnp.where(den > 0.0, den, 1.0)
        t = A[:, 1:2]
        return (jnp.dot(S / dsafe, vWm) + (t / dsafe) * vbr)

    fm = finish(Sx[0], Ax[0])
    rm = finish(Sx[1], Ax[1])
    fm_out[...] = fm
    rm_out[...] = rm

    fav = fa[...]
    rav = ra[...]
    m = jnp.maximum(jnp.max(fav), jnp.max(rav))
    wf = jnp.exp(fav - m)
    wr = jnp.exp(rav - m)
    sw = jnp.sum(wf) + jnp.sum(wr)
    vp = (jnp.sum(wf * fm, axis=0, keepdims=True)
          + jnp.sum(wr * rm, axis=0, keepdims=True)) / sw

    h = jnp.dot(vp, imW1[...]) + imb1[...]
    av = ima[0, 0]
    h = jnp.where(h >= 0.0, h, av * h)
    iv = jnp.dot(h, imW2[...]) + imb2[...]
    zi = z[...] + iv
    mu = jnp.mean(zi, axis=-1, keepdims=True)
    var = jnp.mean((zi - mu) ** 2, axis=-1, keepdims=True)
    zr_out[...] = (zi - mu) * lax.rsqrt(var + 1e-5) * lng[...] + lnb[...]


def _stage_d(Sx, Ax, fa, ra, z, p):
    return pl.pallas_call(
        _stage_d_body,
        out_shape=(
            jax.ShapeDtypeStruct((1, P), jnp.float32),
            jax.ShapeDtypeStruct((NN, D), jnp.float32),
            jax.ShapeDtypeStruct((NN, D), jnp.float32),
        ),
    )(Sx, Ax, p["v_W"], p["v_b"].reshape(1, D), fa, ra, z,
      p["im_W1"], p["im_b1"].reshape(1, P), p["im_a"].reshape(1, 1),
      p["im_W2"], p["im_b2"].reshape(1, P), p["ln_g"].reshape(1, P),
      p["ln_b"].reshape(1, P))


def kernel(target_features, form_features, role_features, form_neighbors,
           role_neighbors, form_binds_ei, form_binds_y, role_binds_ei,
           role_binds_y, drug_features, params):
    p = params
    tgt = target_features.reshape(1, P)
    z, scores2d = _stage_a(tgt, p, drug_features)
    scores = scores2d.reshape(ND)

    didx2 = jnp.stack([form_binds_ei[1], role_binds_ei[1]])
    src2 = jnp.stack([form_binds_ei[0], role_binds_ei[0]])
    y2 = jnp.stack([form_binds_y, role_binds_y])
    Sx, Ax = _stage_b(scores, didx2, src2, y2, drug_features)

    fa, ra = _stage_c(form_features, role_features, z, p)
    zr, fm, rm = _stage_d(Sx, Ax, fa, ra, z, p)
    return (zr.reshape(P), fm, rm)
